# Initial kernel scaffold; baseline (speedup 1.0000x reference)
import jax
import jax.numpy as jnp
from jax.experimental import pallas as pl


def kernel(inputs, gating_weight, total_token_num):
    raise NotImplementedError("write your pallas kernel here")


# speedup vs baseline: 1.3324x; 1.3324x over previous
import functools

import jax
import jax.numpy as jnp
from jax.experimental import pallas as pl
from jax.experimental.pallas import tpu as pltpu


def _routing_kernel(x_ref, w_ref, e1_ref, e2_ref, p1_ref, p2_ref,
                    g1_ref, g2_ref, aux_ref, *, S, E, C):
    x = x_ref[0]
    w = w_ref[...]
    logits = jnp.dot(x, w, preferred_element_type=jnp.float32)

    m = jnp.max(logits, axis=-1, keepdims=True)
    ex = jnp.exp(logits - m)
    raw = ex / jnp.sum(ex, axis=-1, keepdims=True)

    iota_e = jax.lax.broadcasted_iota(jnp.int32, (S, E), 1).astype(jnp.float32)

    mx1 = jnp.max(raw, axis=-1, keepdims=True)
    e1 = jnp.min(jnp.where(raw == mx1, iota_e, jnp.float32(E)),
                 axis=-1, keepdims=True)
    oh1 = (iota_e == e1).astype(jnp.float32)

    raw2 = raw * (1.0 - oh1)
    mx2 = jnp.max(raw2, axis=-1, keepdims=True)
    e2 = jnp.min(jnp.where(raw2 == mx2, iota_e, jnp.float32(E)),
                 axis=-1, keepdims=True)
    oh2 = (iota_e == e2).astype(jnp.float32)

    def _cumsum0(x):
        k = 1
        while k < x.shape[0]:
            shifted = jnp.concatenate(
                [jnp.zeros((k, x.shape[1]), x.dtype), x[:-k]], axis=0)
            x = x + shifted
            k *= 2
        return x

    cs1 = _cumsum0(oh1)
    cs2 = _cumsum0(oh2)
    pos1 = jnp.sum((cs1 - oh1) * oh1, axis=-1, keepdims=True)
    total1 = jnp.sum(oh1, axis=0, keepdims=True)
    cap1 = jnp.minimum(total1, jnp.float32(C))
    pos2 = (jnp.sum((cs2 - oh2) * oh2, axis=-1, keepdims=True)
            + jnp.sum(oh2 * cap1, axis=-1, keepdims=True))

    keep1 = (pos1 < C).astype(jnp.float32)
    keep2 = (pos2 < C).astype(jnp.float32)
    g1 = mx1 * keep1
    g2 = mx2 * keep2
    denom = g1 + g2
    denom = jnp.where(denom > 0, denom, 1.0)
    g1n = g1 / denom
    g2n = g2 / denom

    e1_ref[0] = e1
    e2_ref[0] = e2
    p1_ref[0] = pos1
    p2_ref[0] = pos2
    g1_ref[0] = g1n
    g2_ref[0] = g2n

    d = jnp.float32(1.0 + 1e-6)
    proxy = (jnp.sum(raw, axis=0, keepdims=True) / S) / d
    dens = (total1 / S) / d
    aux_g = jnp.sum(proxy * dens)
    aux_ref[0] = jnp.full((8, 128), aux_g, dtype=jnp.float32)


def _construct_kernel(e1_ref, e2_ref, p1_ref, p2_ref, g1_ref, g2_ref,
                      comb_ref, disp_ref, *, SB, E, C):
    e1 = e1_ref[0]
    e2 = e2_ref[0]
    p1 = p1_ref[0]
    p2 = p2_ref[0]
    g1 = g1_ref[0]
    g2 = g2_ref[0]

    iota_e = jax.lax.broadcasted_iota(jnp.int32, (SB, E), 1).astype(jnp.float32)
    iota_c = jax.lax.broadcasted_iota(jnp.int32, (SB, C), 1).astype(jnp.float32)

    ohe1 = (iota_e == e1).astype(jnp.float32)
    ohe2 = (iota_e == e2).astype(jnp.float32)
    gc1 = g1 * (iota_c == p1).astype(jnp.float32)
    gc2 = g2 * (iota_c == p2).astype(jnp.float32)

    comb = ohe1[:, :, None] * gc1[:, None, :] + ohe2[:, :, None] * gc2[:, None, :]
    comb_ref[0] = comb
    disp_ref[0] = (comb != 0.0).astype(jnp.float32)


def kernel(inputs, gating_weight, total_token_num):
    G, S, M = inputs.shape
    E = gating_weight.shape[1]
    C = 256

    route = pl.pallas_call(
        functools.partial(_routing_kernel, S=S, E=E, C=C),
        grid=(G,),
        in_specs=[
            pl.BlockSpec((1, S, M), lambda g: (g, 0, 0)),
            pl.BlockSpec((M, E), lambda g: (0, 0)),
        ],
        out_specs=[pl.BlockSpec((1, S, 1), lambda g: (g, 0, 0))] * 6 + [
            pl.BlockSpec((1, 8, 128), lambda g: (g, 0, 0)),
        ],
        out_shape=[jax.ShapeDtypeStruct((G, S, 1), jnp.float32)] * 6 + [
            jax.ShapeDtypeStruct((G, 8, 128), jnp.float32),
        ],
    )
    e1, e2, p1, p2, g1, g2, auxp = route(inputs, gating_weight)

    SB = 256
    NSB = S // SB
    tok_spec = pl.BlockSpec((1, SB, 1), lambda g, sb: (g, sb, 0))
    construct = pl.pallas_call(
        functools.partial(_construct_kernel, SB=SB, E=E, C=C),
        grid=(G, NSB),
        in_specs=[tok_spec] * 6,
        out_specs=[
            pl.BlockSpec((1, SB, E, C), lambda g, sb: (g, sb, 0, 0)),
            pl.BlockSpec((1, SB, E, C), lambda g, sb: (g, sb, 0, 0)),
        ],
        out_shape=[
            jax.ShapeDtypeStruct((G, S, E, C), jnp.float32),
            jax.ShapeDtypeStruct((G, S, E, C), jnp.float32),
        ],
    )
    combine_tensor, dispatch_mask = construct(e1, e2, p1, p2, g1, g2)

    aux_loss = jnp.sum(auxp[:, 0, 0]) * jnp.float32(E) / jnp.float32(G)
    return combine_tensor, dispatch_mask, aux_loss
